# Initial kernel scaffold; baseline (speedup 1.0000x reference)
import jax
import jax.numpy as jnp
from jax.experimental import pallas as pl


def kernel(x, timestamps, edge_times, edge_feat, neighbors, w_t, b_t, W1, b1, W2, b2):
    raise NotImplementedError("write your pallas kernel here")


# speedup vs baseline: 1.2629x; 1.2629x over previous
import functools

import jax
import jax.numpy as jnp
from jax import lax
from jax.experimental import pallas as pl
from jax.experimental.pallas import tpu as pltpu
from jax.experimental.pallas import tpu_sc as plsc

N, D, K, DE = 10000, 128, 20, 16

NC, NS, L = 2, 16, 16
NW = NC * NS
NPAD = 10240
PW = NPAD // NW
BN = 8
NG = 2
RG = BN * K // NG
NB = PW // BN

@functools.cache
def _make_sc_gather_sum():
    mesh = plsc.VectorSubcoreMesh(core_axis_name="c", subcore_axis_name="s",
                                  num_cores=NC, num_subcores=NS)

    @functools.partial(
        pl.kernel,
        out_type=jax.ShapeDtypeStruct((NPAD, D), jnp.float32),
        mesh=mesh,
        scratch_types=[
            pltpu.VMEM((NB, NG, RG), jnp.int32),
            pltpu.VMEM((NG, RG, D), jnp.float32),
            pltpu.VMEM((BN, D), jnp.float32),
            pltpu.SemaphoreType.DMA,
        ],
    )
    def _sc_gather_sum(x_hbm, idx_hbm, out_hbm, idx_v, rows_v, acc_v, sem):
        wid = lax.axis_index("s") * NC + lax.axis_index("c")
        pltpu.sync_copy(idx_hbm.at[wid], idx_v)

        def body(b, carry):
            handles = [pltpu.async_copy(x_hbm.at[idx_v.at[b, g]],
                                        rows_v.at[g], sem)
                       for g in range(NG)]
            for h in handles:
                h.wait()
            for i in range(BN):
                for v in range(D // L):
                    r0 = i * K
                    s = rows_v[r0 // RG, r0 % RG, pl.ds(v * L, L)]
                    for k in range(1, K):
                        r = r0 + k
                        s = s + rows_v[r // RG, r % RG, pl.ds(v * L, L)]
                    acc_v[i, pl.ds(v * L, L)] = s
            pltpu.sync_copy(acc_v, out_hbm.at[pl.ds(wid * PW + b * BN, BN)])
            return carry

        lax.fori_loop(0, NB, body, 0)

    return _sc_gather_sum


def _tc_body(ts_ref, et_ref, ef_ref, x_ref, snf_ref, wt_ref, bt_ref,
             w1a_ref, w1b_ref, w1cr_ref, b1_ref, w2a_ref, w2b_ref, w2c_ref,
             b2_ref, out_ref):
    w = wt_ref[...]
    b = bt_ref[...]
    dt = ts_ref[...] - et_ref[...]
    s_te = jnp.cos(dt[:, 0][:, None] * w + b)
    for k in range(1, K):
        s_te = s_te + jnp.cos(dt[:, k][:, None] * w + b)
    acc = lax.dot_general(snf_ref[...], w1a_ref[...], (((1,), (0,)), ((), ())),
                          preferred_element_type=jnp.float32)
    acc += lax.dot_general(s_te, w1b_ref[...], (((1,), (0,)), ((), ())),
                           preferred_element_type=jnp.float32)
    acc += lax.dot_general(ef_ref[...], w1cr_ref[...], (((1,), (0,)), ((), ())),
                           preferred_element_type=jnp.float32)
    h = jnp.maximum(acc + K * b1_ref[...], 0.0)
    out = lax.dot_general(h, w2a_ref[...], (((1,), (0,)), ((), ())),
                          preferred_element_type=jnp.float32)
    out += lax.dot_general(x_ref[...], w2b_ref[...], (((1,), (0,)), ((), ())),
                           preferred_element_type=jnp.float32)
    src = jnp.cos(b)
    out += lax.dot_general(src, w2c_ref[...], (((1,), (0,)), ((), ())),
                           preferred_element_type=jnp.float32)
    out_ref[...] = out + b2_ref[...]


def _tc_combine(ts2, et, ef_flat, x, snf, wt2, bt2, w1a, w1b, w1cr, b12,
                w2a, w2b, w2c, b22, interpret=False):
    B = 256
    grid = (pl.cdiv(N, B),)
    row = lambda i: (i, 0)
    full = lambda i: (0, 0)
    return pl.pallas_call(
        _tc_body,
        grid=grid,
        in_specs=[
            pl.BlockSpec((B, 1), row),
            pl.BlockSpec((B, K), row),
            pl.BlockSpec((B, K * DE), row),
            pl.BlockSpec((B, D), row),
            pl.BlockSpec((B, D), row),
            pl.BlockSpec((1, D), full),
            pl.BlockSpec((1, D), full),
            pl.BlockSpec((D, D), full),
            pl.BlockSpec((D, D), full),
            pl.BlockSpec((K * DE, D), full),
            pl.BlockSpec((1, D), full),
            pl.BlockSpec((D, D), full),
            pl.BlockSpec((D, D), full),
            pl.BlockSpec((D, D), full),
            pl.BlockSpec((1, D), full),
        ],
        out_specs=pl.BlockSpec((B, D), row),
        out_shape=jax.ShapeDtypeStruct((N, D), jnp.float32),
        interpret=interpret,
    )(ts2, et, ef_flat, x, snf, wt2, bt2, w1a, w1b, w1cr, b12,
      w2a, w2b, w2c, b22)


def kernel(x, timestamps, edge_times, edge_feat, neighbors, w_t, b_t,
           W1, b1, W2, b2):
    nbr = neighbors.astype(jnp.int32)
    nbr_pad = jnp.pad(nbr, ((0, NPAD - N), (0, 0)))
    idx3 = nbr_pad.reshape(NW, NB, NG, RG)
    snf = _make_sc_gather_sum()(x, idx3)

    ts2 = timestamps.reshape(N, 1)
    ef_flat = edge_feat.reshape(N, K * DE)
    w1a, w1b, w1c = W1[:D], W1[D:2 * D], W1[2 * D:]
    w1cr = jnp.tile(w1c, (K, 1))
    w2a, w2b, w2c = W2[:D], W2[D:2 * D], W2[2 * D:]
    return _tc_combine(ts2, edge_times, ef_flat, x, snf[:N],
                       w_t.reshape(1, D), b_t.reshape(1, D),
                       w1a, w1b, w1cr, b1.reshape(1, D),
                       w2a, w2b, w2c, b2.reshape(1, D))


# speedup vs baseline: 1.3929x; 1.1029x over previous
import functools

import jax
import jax.numpy as jnp
from jax import lax
from jax.experimental import pallas as pl
from jax.experimental.pallas import tpu as pltpu
from jax.experimental.pallas import tpu_sc as plsc

N, D, K, DE = 10000, 128, 20, 16

NC, NS, L = 2, 16, 16
NW = NC * NS
NPAD = 10240
PW = NPAD // NW
BN = 8
NG = 2
RG = BN * K // NG
NB = PW // BN

@functools.cache
def _make_sc_gather_sum():
    mesh = plsc.VectorSubcoreMesh(core_axis_name="c", subcore_axis_name="s",
                                  num_cores=NC, num_subcores=NS)

    @functools.partial(
        pl.kernel,
        out_type=jax.ShapeDtypeStruct((NPAD, D), jnp.float32),
        mesh=mesh,
        scratch_types=[
            pltpu.VMEM((NB, NG, RG), jnp.int32),
            pltpu.VMEM((2, NG, RG, D), jnp.float32),
            pltpu.VMEM((2, BN, D), jnp.float32),
            pltpu.SemaphoreType.DMA,
            pltpu.SemaphoreType.DMA,
        ],
    )
    def _sc_gather_sum(x_hbm, idx_hbm, out_hbm, idx_v, rows_v, acc_v,
                       sem0, sem1):
        wid = lax.axis_index("s") * NC + lax.axis_index("c")
        pltpu.sync_copy(idx_hbm.at[wid], idx_v)
        sems = [sem0, sem1]

        def fire(b, p):
            for g in range(NG):
                pltpu.async_copy(x_hbm.at[idx_v.at[b, g]],
                                 rows_v.at[p, g], sems[p])

        def drain(p):
            for g in range(NG):
                pltpu.make_async_copy(x_hbm.at[pl.ds(0, RG)],
                                      rows_v.at[p, g], sems[p]).wait()

        def accum_write(b, p):
            for i in range(BN):
                for v in range(D // L):
                    r0 = i * K
                    s = rows_v[p, r0 // RG, r0 % RG, pl.ds(v * L, L)]
                    for k in range(1, K):
                        r = r0 + k
                        s = s + rows_v[p, r // RG, r % RG, pl.ds(v * L, L)]
                    acc_v[p, i, pl.ds(v * L, L)] = s
            pltpu.sync_copy(acc_v.at[p],
                            out_hbm.at[pl.ds(wid * PW + b * BN, BN)])

        fire(0, 0)

        def body(j, carry):
            b0 = 2 * j
            fire(b0 + 1, 1)
            drain(0)
            accum_write(b0, 0)

            @pl.when(j < NB // 2 - 1)
            def _():
                fire(b0 + 2, 0)

            drain(1)
            accum_write(b0 + 1, 1)
            return carry

        lax.fori_loop(0, NB // 2, body, 0)

    return _sc_gather_sum


def _tc_body(ts_ref, et_ref, ef_ref, x_ref, snf_ref, wt_ref, bt_ref,
             w1a_ref, w1b_ref, w1cr_ref, b1_ref, w2a_ref, w2b_ref, w2c_ref,
             b2_ref, out_ref):
    w = wt_ref[...]
    b = bt_ref[...]
    dt = ts_ref[...] - et_ref[...]
    s_te = jnp.cos(dt[:, 0][:, None] * w + b)
    for k in range(1, K):
        s_te = s_te + jnp.cos(dt[:, k][:, None] * w + b)
    acc = lax.dot_general(snf_ref[...], w1a_ref[...], (((1,), (0,)), ((), ())),
                          preferred_element_type=jnp.float32)
    acc += lax.dot_general(s_te, w1b_ref[...], (((1,), (0,)), ((), ())),
                           preferred_element_type=jnp.float32)
    acc += lax.dot_general(ef_ref[...], w1cr_ref[...], (((1,), (0,)), ((), ())),
                           preferred_element_type=jnp.float32)
    h = jnp.maximum(acc + K * b1_ref[...], 0.0)
    out = lax.dot_general(h, w2a_ref[...], (((1,), (0,)), ((), ())),
                          preferred_element_type=jnp.float32)
    out += lax.dot_general(x_ref[...], w2b_ref[...], (((1,), (0,)), ((), ())),
                           preferred_element_type=jnp.float32)
    src = jnp.cos(b)
    out += lax.dot_general(src, w2c_ref[...], (((1,), (0,)), ((), ())),
                           preferred_element_type=jnp.float32)
    out_ref[...] = out + b2_ref[...]


def _tc_combine(ts2, et, ef_flat, x, snf, wt2, bt2, w1a, w1b, w1cr, b12,
                w2a, w2b, w2c, b22, interpret=False):
    B = 256
    grid = (pl.cdiv(N, B),)
    row = lambda i: (i, 0)
    full = lambda i: (0, 0)
    return pl.pallas_call(
        _tc_body,
        grid=grid,
        in_specs=[
            pl.BlockSpec((B, 1), row),
            pl.BlockSpec((B, K), row),
            pl.BlockSpec((B, K * DE), row),
            pl.BlockSpec((B, D), row),
            pl.BlockSpec((B, D), row),
            pl.BlockSpec((1, D), full),
            pl.BlockSpec((1, D), full),
            pl.BlockSpec((D, D), full),
            pl.BlockSpec((D, D), full),
            pl.BlockSpec((K * DE, D), full),
            pl.BlockSpec((1, D), full),
            pl.BlockSpec((D, D), full),
            pl.BlockSpec((D, D), full),
            pl.BlockSpec((D, D), full),
            pl.BlockSpec((1, D), full),
        ],
        out_specs=pl.BlockSpec((B, D), row),
        out_shape=jax.ShapeDtypeStruct((N, D), jnp.float32),
        interpret=interpret,
    )(ts2, et, ef_flat, x, snf, wt2, bt2, w1a, w1b, w1cr, b12,
      w2a, w2b, w2c, b22)


def kernel(x, timestamps, edge_times, edge_feat, neighbors, w_t, b_t,
           W1, b1, W2, b2):
    nbr = neighbors.astype(jnp.int32)
    nbr_pad = jnp.pad(nbr, ((0, NPAD - N), (0, 0)))
    idx3 = nbr_pad.reshape(NW, NB, NG, RG)
    snf = _make_sc_gather_sum()(x, idx3)

    ts2 = timestamps.reshape(N, 1)
    ef_flat = edge_feat.reshape(N, K * DE)
    w1a, w1b, w1c = W1[:D], W1[D:2 * D], W1[2 * D:]
    w1cr = jnp.tile(w1c, (K, 1))
    w2a, w2b, w2c = W2[:D], W2[D:2 * D], W2[2 * D:]
    return _tc_combine(ts2, edge_times, ef_flat, x, snf[:N],
                       w_t.reshape(1, D), b_t.reshape(1, D),
                       w1a, w1b, w1cr, b1.reshape(1, D),
                       w2a, w2b, w2c, b2.reshape(1, D))


# speedup vs baseline: 2.1889x; 1.5714x over previous
import functools

import jax
import jax.numpy as jnp
from jax import lax
from jax.experimental import pallas as pl
from jax.experimental.pallas import tpu as pltpu
from jax.experimental.pallas import tpu_sc as plsc

N, D, K, DE = 10000, 128, 20, 16

NC, NS, L = 2, 16, 16
NW = NC * NS
NPAD = 10240
PW = NPAD // NW
BN = 8
NG = 2
RG = BN * K // NG
NB = PW // BN

@functools.cache
def _make_sc_gather_sum():
    mesh = plsc.VectorSubcoreMesh(core_axis_name="c", subcore_axis_name="s",
                                  num_cores=NC, num_subcores=NS)

    @functools.partial(
        pl.kernel,
        out_type=jax.ShapeDtypeStruct((NPAD, D), jnp.float32),
        mesh=mesh,
        scratch_types=[
            pltpu.VMEM((NB, NG, RG), jnp.int32),
            pltpu.VMEM((2, NG, RG, D), jnp.float32),
            pltpu.VMEM((2, BN, D), jnp.float32),
            pltpu.SemaphoreType.DMA,
            pltpu.SemaphoreType.DMA,
        ],
    )
    def _sc_gather_sum(x_hbm, idx_hbm, out_hbm, idx_v, rows_v, acc_v,
                       sem0, sem1):
        wid = lax.axis_index("s") * NC + lax.axis_index("c")
        pltpu.sync_copy(idx_hbm.at[wid], idx_v)
        sems = [sem0, sem1]

        def fire(b, p):
            for g in range(NG):
                pltpu.async_copy(x_hbm.at[idx_v.at[b, g]],
                                 rows_v.at[p, g], sems[p])

        def drain(p):
            for g in range(NG):
                pltpu.make_async_copy(x_hbm.at[pl.ds(0, RG)],
                                      rows_v.at[p, g], sems[p]).wait()

        def accum_write(b, p):
            for i in range(BN):
                for v in range(D // L):
                    r0 = i * K
                    s = rows_v[p, r0 // RG, r0 % RG, pl.ds(v * L, L)]
                    for k in range(1, K):
                        r = r0 + k
                        s = s + rows_v[p, r // RG, r % RG, pl.ds(v * L, L)]
                    acc_v[p, i, pl.ds(v * L, L)] = s
            pltpu.sync_copy(acc_v.at[p],
                            out_hbm.at[pl.ds(wid * PW + b * BN, BN)])

        fire(0, 0)

        def body(j, carry):
            b0 = 2 * j
            fire(b0 + 1, 1)
            drain(0)
            accum_write(b0, 0)

            @pl.when(j < NB // 2 - 1)
            def _():
                fire(b0 + 2, 0)

            drain(1)
            accum_write(b0 + 1, 1)
            return carry

        lax.fori_loop(0, NB // 2, body, 0)

    return _sc_gather_sum


def _tc1_body(ts_ref, et_ref, ef_ref, x_ref, wt_ref, bt_ref,
              w1b_ref, w1cr_ref, b1_ref, w2b_ref, w2c_ref, b2_ref,
              a_ref, p_ref):
    w = wt_ref[...]
    b = bt_ref[...]
    dt = ts_ref[...] - et_ref[...]
    s_te = jnp.cos(dt[:, 0][:, None] * w + b)
    for k in range(1, K):
        s_te = s_te + jnp.cos(dt[:, k][:, None] * w + b)
    a = lax.dot_general(s_te, w1b_ref[...], (((1,), (0,)), ((), ())),
                        preferred_element_type=jnp.float32)
    a += lax.dot_general(ef_ref[...], w1cr_ref[...], (((1,), (0,)), ((), ())),
                         preferred_element_type=jnp.float32)
    a_ref[...] = a + K * b1_ref[...]
    p = lax.dot_general(x_ref[...], w2b_ref[...], (((1,), (0,)), ((), ())),
                        preferred_element_type=jnp.float32)
    src = jnp.cos(b)
    p += lax.dot_general(src, w2c_ref[...], (((1,), (0,)), ((), ())),
                         preferred_element_type=jnp.float32)
    p_ref[...] = p + b2_ref[...]


def _tc1(ts2, et, ef_flat, x, wt2, bt2, w1b, w1cr, b12, w2b, w2c, b22,
         interpret=False):
    B = 256
    grid = (pl.cdiv(N, B),)
    row = lambda i: (i, 0)
    full = lambda i: (0, 0)
    return pl.pallas_call(
        _tc1_body,
        grid=grid,
        in_specs=[
            pl.BlockSpec((B, 1), row),
            pl.BlockSpec((B, K), row),
            pl.BlockSpec((B, K * DE), row),
            pl.BlockSpec((B, D), row),
            pl.BlockSpec((1, D), full),
            pl.BlockSpec((1, D), full),
            pl.BlockSpec((D, D), full),
            pl.BlockSpec((K * DE, D), full),
            pl.BlockSpec((1, D), full),
            pl.BlockSpec((D, D), full),
            pl.BlockSpec((D, D), full),
            pl.BlockSpec((1, D), full),
        ],
        out_specs=[pl.BlockSpec((B, D), row), pl.BlockSpec((B, D), row)],
        out_shape=[jax.ShapeDtypeStruct((N, D), jnp.float32),
                   jax.ShapeDtypeStruct((N, D), jnp.float32)],
        interpret=interpret,
    )(ts2, et, ef_flat, x, wt2, bt2, w1b, w1cr, b12, w2b, w2c, b22)


def _tc2_body(snf_ref, a_ref, p_ref, w1a_ref, w2a_ref, out_ref):
    acc = lax.dot_general(snf_ref[...], w1a_ref[...], (((1,), (0,)), ((), ())),
                          preferred_element_type=jnp.float32)
    h = jnp.maximum(acc + a_ref[...], 0.0)
    out = lax.dot_general(h, w2a_ref[...], (((1,), (0,)), ((), ())),
                          preferred_element_type=jnp.float32)
    out_ref[...] = out + p_ref[...]


def _tc2(snf, a, p, w1a, w2a, interpret=False):
    B = 256
    grid = (pl.cdiv(N, B),)
    row = lambda i: (i, 0)
    full = lambda i: (0, 0)
    return pl.pallas_call(
        _tc2_body,
        grid=grid,
        in_specs=[
            pl.BlockSpec((B, D), row),
            pl.BlockSpec((B, D), row),
            pl.BlockSpec((B, D), row),
            pl.BlockSpec((D, D), full),
            pl.BlockSpec((D, D), full),
        ],
        out_specs=pl.BlockSpec((B, D), row),
        out_shape=jax.ShapeDtypeStruct((N, D), jnp.float32),
        interpret=interpret,
    )(snf, a, p, w1a, w2a)


def kernel(x, timestamps, edge_times, edge_feat, neighbors, w_t, b_t,
           W1, b1, W2, b2):
    nbr = neighbors.astype(jnp.int32)
    nbr_pad = jnp.pad(nbr, ((0, NPAD - N), (0, 0)))
    idx3 = nbr_pad.reshape(NW, NB, NG, RG)
    snf = _make_sc_gather_sum()(x, idx3)

    ts2 = timestamps.reshape(N, 1)
    ef_flat = edge_feat.reshape(N, K * DE)
    w1a, w1b, w1c = W1[:D], W1[D:2 * D], W1[2 * D:]
    w1cr = jnp.tile(w1c, (K, 1))
    w2a, w2b, w2c = W2[:D], W2[D:2 * D], W2[2 * D:]
    a, p = _tc1(ts2, edge_times, ef_flat, x,
                w_t.reshape(1, D), b_t.reshape(1, D),
                w1b, w1cr, b1.reshape(1, D),
                w2b, w2c, b2.reshape(1, D))
    return _tc2(snf[:N], a, p, w1a, w2a)


# speedup vs baseline: 2.2084x; 1.0089x over previous
import functools

import jax
import jax.numpy as jnp
from jax import lax
from jax.experimental import pallas as pl
from jax.experimental.pallas import tpu as pltpu
from jax.experimental.pallas import tpu_sc as plsc

N, D, K, DE = 10000, 128, 20, 16

NC, NS, L = 2, 16, 16
NW = NC * NS
NPAD = 10240
PW = NPAD // NW
BN = 8
NG = 2
RG = BN * K // NG
NB = PW // BN

@functools.cache
def _make_sc_gather_sum():
    mesh = plsc.VectorSubcoreMesh(core_axis_name="c", subcore_axis_name="s",
                                  num_cores=NC, num_subcores=NS)

    @functools.partial(
        pl.kernel,
        out_type=jax.ShapeDtypeStruct((NPAD, D), jnp.float32),
        mesh=mesh,
        scratch_types=[
            pltpu.VMEM((NB, NG, RG), jnp.int32),
            pltpu.VMEM((2, NG, RG, D), jnp.float32),
            pltpu.VMEM((2, BN, D), jnp.float32),
            pltpu.SemaphoreType.DMA,
            pltpu.SemaphoreType.DMA,
        ],
    )
    def _sc_gather_sum(x_hbm, idx_hbm, out_hbm, idx_v, rows_v, acc_v,
                       sem0, sem1):
        wid = lax.axis_index("s") * NC + lax.axis_index("c")
        pltpu.sync_copy(idx_hbm.at[wid], idx_v)
        sems = [sem0, sem1]

        def fire(b, p):
            for g in range(NG):
                pltpu.async_copy(x_hbm.at[idx_v.at[b, g]],
                                 rows_v.at[p, g], sems[p])

        def drain(p):
            for g in range(NG):
                pltpu.make_async_copy(x_hbm.at[pl.ds(0, RG)],
                                      rows_v.at[p, g], sems[p]).wait()

        def accum_write(b, p):
            for i in range(BN):
                for v in range(D // L):
                    r0 = i * K
                    s = rows_v[p, r0 // RG, r0 % RG, pl.ds(v * L, L)]
                    for k in range(1, K):
                        r = r0 + k
                        s = s + rows_v[p, r // RG, r % RG, pl.ds(v * L, L)]
                    acc_v[p, i, pl.ds(v * L, L)] = s
            pltpu.sync_copy(acc_v.at[p],
                            out_hbm.at[pl.ds(wid * PW + b * BN, BN)])

        fire(0, 0)

        def body(j, carry):
            b0 = 2 * j
            fire(b0 + 1, 1)
            drain(0)
            accum_write(b0, 0)

            @pl.when(j < NB // 2 - 1)
            def _():
                fire(b0 + 2, 0)

            drain(1)
            accum_write(b0 + 1, 1)
            return carry

        lax.fori_loop(0, NB // 2, body, 0)

    return _sc_gather_sum


def _tc1_body(ts_ref, et_ref, ef_ref, x_ref, wt_ref, bt_ref,
              w1b_ref, w1cr_ref, b1_ref, w2b_ref, w2c_ref, b2_ref,
              a_ref, p_ref):
    w = wt_ref[...]
    b = bt_ref[...]
    dt = ts_ref[...] - et_ref[...]
    s_te = jnp.cos(dt[:, 0][:, None] * w + b)
    for k in range(1, K):
        s_te = s_te + jnp.cos(dt[:, k][:, None] * w + b)
    a = lax.dot_general(s_te, w1b_ref[...], (((1,), (0,)), ((), ())),
                        preferred_element_type=jnp.float32)
    a += lax.dot_general(ef_ref[...], w1cr_ref[...], (((1,), (0,)), ((), ())),
                         preferred_element_type=jnp.float32)
    a_ref[...] = a + K * b1_ref[...]
    p = lax.dot_general(x_ref[...], w2b_ref[...], (((1,), (0,)), ((), ())),
                        preferred_element_type=jnp.float32)
    src = jnp.cos(b)
    p += lax.dot_general(src, w2c_ref[...], (((1,), (0,)), ((), ())),
                         preferred_element_type=jnp.float32)
    p_ref[...] = p + b2_ref[...]


def _tc1(ts2, et, ef_flat, x, wt2, bt2, w1b, w1cr, b12, w2b, w2c, b22,
         interpret=False):
    B = 256
    grid = (pl.cdiv(N, B),)
    row = lambda i: (i, 0)
    full = lambda i: (0, 0)
    return pl.pallas_call(
        _tc1_body,
        grid=grid,
        in_specs=[
            pl.BlockSpec((B, 1), row),
            pl.BlockSpec((B, K), row),
            pl.BlockSpec((B, K * DE), row),
            pl.BlockSpec((B, D), row),
            pl.BlockSpec((1, D), full),
            pl.BlockSpec((1, D), full),
            pl.BlockSpec((D, D), full),
            pl.BlockSpec((K * DE, D), full),
            pl.BlockSpec((1, D), full),
            pl.BlockSpec((D, D), full),
            pl.BlockSpec((D, D), full),
            pl.BlockSpec((1, D), full),
        ],
        out_specs=[pl.BlockSpec((B, D), row), pl.BlockSpec((B, D), row)],
        out_shape=[jax.ShapeDtypeStruct((N, D), jnp.float32),
                   jax.ShapeDtypeStruct((N, D), jnp.float32)],
        interpret=interpret,
    )(ts2, et, ef_flat, x, wt2, bt2, w1b, w1cr, b12, w2b, w2c, b22)


def _tc2_body(snf_ref, a_ref, p_ref, w1a_ref, w2a_ref, out_ref):
    acc = lax.dot_general(snf_ref[...], w1a_ref[...], (((1,), (0,)), ((), ())),
                          preferred_element_type=jnp.float32)
    h = jnp.maximum(acc + a_ref[...], 0.0)
    out = lax.dot_general(h, w2a_ref[...], (((1,), (0,)), ((), ())),
                          preferred_element_type=jnp.float32)
    out_ref[...] = out + p_ref[...]


def _tc2(snf, a, p, w1a, w2a, interpret=False):
    B = 256
    grid = (pl.cdiv(N, B),)
    row = lambda i: (i, 0)
    full = lambda i: (0, 0)
    return pl.pallas_call(
        _tc2_body,
        grid=grid,
        in_specs=[
            pl.BlockSpec((B, D), row),
            pl.BlockSpec((B, D), row),
            pl.BlockSpec((B, D), row),
            pl.BlockSpec((D, D), full),
            pl.BlockSpec((D, D), full),
        ],
        out_specs=pl.BlockSpec((B, D), row),
        out_shape=jax.ShapeDtypeStruct((N, D), jnp.float32),
        interpret=interpret,
    )(snf, a, p, w1a, w2a)


def kernel(x, timestamps, edge_times, edge_feat, neighbors, w_t, b_t,
           W1, b1, W2, b2):
    nbr = neighbors.astype(jnp.int32)
    nbr_pad = jnp.pad(nbr, ((0, NPAD - N), (0, 0)))
    idx3 = nbr_pad.reshape(NW, NB, NG, RG)
    snf = _make_sc_gather_sum()(x, idx3)

    ts2 = timestamps.reshape(N, 1)
    ef_flat = edge_feat.reshape(N, K * DE)
    w1a, w1b, w1c = W1[:D], W1[D:2 * D], W1[2 * D:]
    w1cr = jnp.tile(w1c, (K, 1))
    w2a, w2b, w2c = W2[:D], W2[D:2 * D], W2[2 * D:]
    a, p = _tc1(ts2, edge_times, ef_flat, x,
                w_t.reshape(1, D), b_t.reshape(1, D),
                w1b, w1cr, b1.reshape(1, D),
                w2b, w2c, b2.reshape(1, D))
    return _tc2(snf, a, p, w1a, w2a)


# speedup vs baseline: 2.2673x; 1.0267x over previous
import functools

import jax
import jax.numpy as jnp
from jax import lax
from jax.experimental import pallas as pl
from jax.experimental.pallas import tpu as pltpu
from jax.experimental.pallas import tpu_sc as plsc

N, D, K, DE = 10000, 128, 20, 16

NC, NS, L = 2, 16, 16
NW = NC * NS
NPAD = 10240
PW = NPAD // NW
BN = 8
NG = 2
RG = BN * K // NG
NB = PW // BN

@functools.cache
def _make_sc_gather_sum():
    mesh = plsc.VectorSubcoreMesh(core_axis_name="c", subcore_axis_name="s",
                                  num_cores=NC, num_subcores=NS)

    @functools.partial(
        pl.kernel,
        out_type=jax.ShapeDtypeStruct((NPAD, D), jnp.float32),
        mesh=mesh,
        scratch_types=[
            pltpu.VMEM((NB, NG, RG), jnp.int32),
            pltpu.VMEM((2, NG, RG, D), jnp.float32),
            pltpu.VMEM((2, BN, D), jnp.float32),
            pltpu.SemaphoreType.DMA,
            pltpu.SemaphoreType.DMA,
        ],
    )
    def _sc_gather_sum(x_hbm, idx_hbm, out_hbm, idx_v, rows_v, acc_v,
                       sem0, sem1):
        wid = lax.axis_index("s") * NC + lax.axis_index("c")
        pltpu.sync_copy(idx_hbm.at[wid], idx_v)
        sems = [sem0, sem1]

        def fire(b, p):
            for g in range(NG):
                pltpu.async_copy(x_hbm.at[idx_v.at[b, g]],
                                 rows_v.at[p, g], sems[p])

        def drain(p):
            for g in range(NG):
                pltpu.make_async_copy(x_hbm.at[pl.ds(0, RG)],
                                      rows_v.at[p, g], sems[p]).wait()

        def accum_write(b, p):
            for i in range(BN):
                for v in range(D // L):
                    r0 = i * K
                    s = rows_v[p, r0 // RG, r0 % RG, pl.ds(v * L, L)]
                    for k in range(1, K):
                        r = r0 + k
                        s = s + rows_v[p, r // RG, r % RG, pl.ds(v * L, L)]
                    acc_v[p, i, pl.ds(v * L, L)] = s
            pltpu.sync_copy(acc_v.at[p],
                            out_hbm.at[pl.ds(wid * PW + b * BN, BN)])

        fire(0, 0)

        def body(j, carry):
            b0 = 2 * j
            fire(b0 + 1, 1)
            drain(0)
            accum_write(b0, 0)

            @pl.when(j < NB // 2 - 1)
            def _():
                fire(b0 + 2, 0)

            drain(1)
            accum_write(b0 + 1, 1)
            return carry

        lax.fori_loop(0, NB // 2, body, 0)

    return _sc_gather_sum


def _tc1_body(ts_ref, et_ref, ef_ref, x_ref, wt_ref, bt_ref,
              w1b_ref, w1cr_ref, b1_ref, w2b_ref, w2c_ref, b2_ref,
              a_ref, p_ref):
    w = wt_ref[...]
    b = bt_ref[...]
    dt = ts_ref[...] - et_ref[...]
    s_te = jnp.cos(dt[:, 0][:, None] * w + b)
    for k in range(1, K):
        s_te = s_te + jnp.cos(dt[:, k][:, None] * w + b)
    a = lax.dot_general(s_te, w1b_ref[...], (((1,), (0,)), ((), ())),
                        preferred_element_type=jnp.float32)
    a += lax.dot_general(ef_ref[...], w1cr_ref[...], (((1,), (0,)), ((), ())),
                         preferred_element_type=jnp.float32)
    a_ref[...] = a + K * b1_ref[...]
    p = lax.dot_general(x_ref[...], w2b_ref[...], (((1,), (0,)), ((), ())),
                        preferred_element_type=jnp.float32)
    src = jnp.cos(b)
    p += lax.dot_general(src, w2c_ref[...], (((1,), (0,)), ((), ())),
                         preferred_element_type=jnp.float32)
    p_ref[...] = p + b2_ref[...]


def _tc1(ts2, et, ef_flat, x, wt2, bt2, w1b, w1cr, b12, w2b, w2c, b22,
         interpret=False):
    B = 256
    grid = (pl.cdiv(N, B),)
    row = lambda i: (i, 0)
    full = lambda i: (0, 0)
    return pl.pallas_call(
        _tc1_body,
        grid=grid,
        in_specs=[
            pl.BlockSpec((B, 1), row),
            pl.BlockSpec((B, K), row),
            pl.BlockSpec((B, K * DE), row),
            pl.BlockSpec((B, D), row),
            pl.BlockSpec((1, D), full),
            pl.BlockSpec((1, D), full),
            pl.BlockSpec((D, D), full),
            pl.BlockSpec((K * DE, D), full),
            pl.BlockSpec((1, D), full),
            pl.BlockSpec((D, D), full),
            pl.BlockSpec((D, D), full),
            pl.BlockSpec((1, D), full),
        ],
        out_specs=[pl.BlockSpec((B, D), row), pl.BlockSpec((B, D), row)],
        out_shape=[jax.ShapeDtypeStruct((N, D), jnp.float32),
                   jax.ShapeDtypeStruct((N, D), jnp.float32)],
        interpret=interpret,
    )(ts2, et, ef_flat, x, wt2, bt2, w1b, w1cr, b12, w2b, w2c, b22)


def _tc2_body(snf_ref, a_ref, p_ref, w1a_ref, w2a_ref, out_ref):
    acc = lax.dot_general(snf_ref[...], w1a_ref[...], (((1,), (0,)), ((), ())),
                          preferred_element_type=jnp.float32)
    h = jnp.maximum(acc + a_ref[...], 0.0)
    out = lax.dot_general(h, w2a_ref[...], (((1,), (0,)), ((), ())),
                          preferred_element_type=jnp.float32)
    out_ref[...] = out + p_ref[...]


def _tc2(snf, a, p, w1a, w2a, interpret=False):
    B = 512
    grid = (pl.cdiv(N, B),)
    row = lambda i: (i, 0)
    full = lambda i: (0, 0)
    return pl.pallas_call(
        _tc2_body,
        grid=grid,
        in_specs=[
            pl.BlockSpec((B, D), row),
            pl.BlockSpec((B, D), row),
            pl.BlockSpec((B, D), row),
            pl.BlockSpec((D, D), full),
            pl.BlockSpec((D, D), full),
        ],
        out_specs=pl.BlockSpec((B, D), row),
        out_shape=jax.ShapeDtypeStruct((N, D), jnp.float32),
        interpret=interpret,
    )(snf, a, p, w1a, w2a)


def kernel(x, timestamps, edge_times, edge_feat, neighbors, w_t, b_t,
           W1, b1, W2, b2):
    nbr = neighbors.astype(jnp.int32)
    nbr_pad = jnp.pad(nbr, ((0, NPAD - N), (0, 0)))
    idx3 = nbr_pad.reshape(NW, NB, NG, RG)
    snf = _make_sc_gather_sum()(x, idx3)

    ts2 = timestamps.reshape(N, 1)
    ef_flat = edge_feat.reshape(N, K * DE)
    w1a, w1b, w1c = W1[:D], W1[D:2 * D], W1[2 * D:]
    w1cr = jnp.tile(w1c, (K, 1))
    w2a, w2b, w2c = W2[:D], W2[D:2 * D], W2[2 * D:]
    a, p = _tc1(ts2, edge_times, ef_flat, x,
                w_t.reshape(1, D), b_t.reshape(1, D),
                w1b, w1cr, b1.reshape(1, D),
                w2b, w2c, b2.reshape(1, D))
    return _tc2(snf, a, p, w1a, w2a)


# speedup vs baseline: 2.2965x; 1.0129x over previous
import functools

import jax
import jax.numpy as jnp
from jax import lax
from jax.experimental import pallas as pl
from jax.experimental.pallas import tpu as pltpu
from jax.experimental.pallas import tpu_sc as plsc

N, D, K, DE = 10000, 128, 20, 16

NC, NS, L = 2, 16, 16
NW = NC * NS
NPAD = 10240
PW = NPAD // NW
BN = 8
NG = 2
RG = BN * K // NG
NB = PW // BN

@functools.cache
def _make_sc_gather_sum():
    mesh = plsc.VectorSubcoreMesh(core_axis_name="c", subcore_axis_name="s",
                                  num_cores=NC, num_subcores=NS)

    @functools.partial(
        pl.kernel,
        out_type=jax.ShapeDtypeStruct((NPAD, D), jnp.float32),
        mesh=mesh,
        scratch_types=[
            pltpu.VMEM((NB, NG, RG), jnp.int32),
            pltpu.VMEM((2, NG, RG, D), jnp.float32),
            pltpu.VMEM((2, BN, D), jnp.float32),
            pltpu.SemaphoreType.DMA,
            pltpu.SemaphoreType.DMA,
        ],
    )
    def _sc_gather_sum(x_hbm, idx_hbm, out_hbm, idx_v, rows_v, acc_v,
                       sem0, sem1):
        wid = lax.axis_index("s") * NC + lax.axis_index("c")
        pltpu.sync_copy(idx_hbm.at[wid], idx_v)
        sems = [sem0, sem1]

        def fire(b, p):
            for g in range(NG):
                pltpu.async_copy(x_hbm.at[idx_v.at[b, g]],
                                 rows_v.at[p, g], sems[p])

        def drain(p):
            for g in range(NG):
                pltpu.make_async_copy(x_hbm.at[pl.ds(0, RG)],
                                      rows_v.at[p, g], sems[p]).wait()

        def accum_write(b, p):
            for i in range(BN):
                for v in range(D // L):
                    r0 = i * K
                    s = rows_v[p, r0 // RG, r0 % RG, pl.ds(v * L, L)]
                    for k in range(1, K):
                        r = r0 + k
                        s = s + rows_v[p, r // RG, r % RG, pl.ds(v * L, L)]
                    acc_v[p, i, pl.ds(v * L, L)] = s
            pltpu.sync_copy(acc_v.at[p],
                            out_hbm.at[pl.ds(wid * PW + b * BN, BN)])

        fire(0, 0)

        def body(j, carry):
            b0 = 2 * j
            fire(b0 + 1, 1)
            drain(0)
            accum_write(b0, 0)

            @pl.when(j < NB // 2 - 1)
            def _():
                fire(b0 + 2, 0)

            drain(1)
            accum_write(b0 + 1, 1)
            return carry

        lax.fori_loop(0, NB // 2, body, 0)

    return _sc_gather_sum


def _tc1_body(ts_ref, et_ref, ef_ref, x_ref, wt_ref, bt_ref,
              w1b_ref, w1cr_ref, b1_ref, w2b_ref, w2c_ref, b2_ref,
              a_ref, p_ref):
    w = wt_ref[...]
    b = bt_ref[...]
    dt = ts_ref[...] - et_ref[...]
    s_te = jnp.cos(dt[:, 0][:, None] * w + b)
    for k in range(1, K):
        s_te = s_te + jnp.cos(dt[:, k][:, None] * w + b)
    a = lax.dot_general(s_te, w1b_ref[...], (((1,), (0,)), ((), ())),
                        preferred_element_type=jnp.float32)
    a += lax.dot_general(ef_ref[...], w1cr_ref[...], (((1,), (0,)), ((), ())),
                         preferred_element_type=jnp.float32)
    a_ref[...] = a + K * b1_ref[...]
    p = lax.dot_general(x_ref[...], w2b_ref[...], (((1,), (0,)), ((), ())),
                        preferred_element_type=jnp.float32)
    src = jnp.cos(b)
    p += lax.dot_general(src, w2c_ref[...], (((1,), (0,)), ((), ())),
                         preferred_element_type=jnp.float32)
    p_ref[...] = p + b2_ref[...]


def _tc1(ts2, et, ef_flat, x, wt2, bt2, w1b, w1cr, b12, w2b, w2c, b22,
         interpret=False):
    B = 256
    grid = (pl.cdiv(N, B),)
    row = lambda i: (i, 0)
    full = lambda i: (0, 0)
    return pl.pallas_call(
        _tc1_body,
        grid=grid,
        in_specs=[
            pl.BlockSpec((B, 1), row),
            pl.BlockSpec((B, K), row),
            pl.BlockSpec((B, K * DE), row),
            pl.BlockSpec((B, D), row),
            pl.BlockSpec((1, D), full),
            pl.BlockSpec((1, D), full),
            pl.BlockSpec((D, D), full),
            pl.BlockSpec((K * DE, D), full),
            pl.BlockSpec((1, D), full),
            pl.BlockSpec((D, D), full),
            pl.BlockSpec((D, D), full),
            pl.BlockSpec((1, D), full),
        ],
        out_specs=[pl.BlockSpec((B, D), row), pl.BlockSpec((B, D), row)],
        out_shape=[jax.ShapeDtypeStruct((N, D), jnp.float32),
                   jax.ShapeDtypeStruct((N, D), jnp.float32)],
        interpret=interpret,
    )(ts2, et, ef_flat, x, wt2, bt2, w1b, w1cr, b12, w2b, w2c, b22)


def _tc2_body(snf_ref, a_ref, p_ref, w1a_ref, w2a_ref, out_ref):
    acc = lax.dot_general(snf_ref[...], w1a_ref[...], (((1,), (0,)), ((), ())),
                          preferred_element_type=jnp.float32)
    h = jnp.maximum(acc + a_ref[...], 0.0)
    out = lax.dot_general(h, w2a_ref[...], (((1,), (0,)), ((), ())),
                          preferred_element_type=jnp.float32)
    out_ref[...] = out + p_ref[...]


def _tc2(snf, a, p, w1a, w2a, interpret=False):
    B = 1024
    grid = (pl.cdiv(N, B),)
    row = lambda i: (i, 0)
    full = lambda i: (0, 0)
    return pl.pallas_call(
        _tc2_body,
        grid=grid,
        in_specs=[
            pl.BlockSpec((B, D), row),
            pl.BlockSpec((B, D), row),
            pl.BlockSpec((B, D), row),
            pl.BlockSpec((D, D), full),
            pl.BlockSpec((D, D), full),
        ],
        out_specs=pl.BlockSpec((B, D), row),
        out_shape=jax.ShapeDtypeStruct((N, D), jnp.float32),
        interpret=interpret,
    )(snf, a, p, w1a, w2a)


def kernel(x, timestamps, edge_times, edge_feat, neighbors, w_t, b_t,
           W1, b1, W2, b2):
    nbr = neighbors.astype(jnp.int32)
    nbr_pad = jnp.pad(nbr, ((0, NPAD - N), (0, 0)))
    idx3 = nbr_pad.reshape(NW, NB, NG, RG)
    snf = _make_sc_gather_sum()(x, idx3)

    ts2 = timestamps.reshape(N, 1)
    ef_flat = edge_feat.reshape(N, K * DE)
    w1a, w1b, w1c = W1[:D], W1[D:2 * D], W1[2 * D:]
    w1cr = jnp.tile(w1c, (K, 1))
    w2a, w2b, w2c = W2[:D], W2[D:2 * D], W2[2 * D:]
    a, p = _tc1(ts2, edge_times, ef_flat, x,
                w_t.reshape(1, D), b_t.reshape(1, D),
                w1b, w1cr, b1.reshape(1, D),
                w2b, w2c, b2.reshape(1, D))
    return _tc2(snf, a, p, w1a, w2a)


# speedup vs baseline: 2.3117x; 1.0066x over previous
import functools

import jax
import jax.numpy as jnp
from jax import lax
from jax.experimental import pallas as pl
from jax.experimental.pallas import tpu as pltpu
from jax.experimental.pallas import tpu_sc as plsc

N, D, K, DE = 10000, 128, 20, 16

NC, NS, L = 2, 16, 16
NW = NC * NS
NPAD = 10240
PW = NPAD // NW
BN = 8
NG = 2
RG = BN * K // NG
NB = PW // BN

@functools.cache
def _make_sc_gather_sum():
    mesh = plsc.VectorSubcoreMesh(core_axis_name="c", subcore_axis_name="s",
                                  num_cores=NC, num_subcores=NS)

    @functools.partial(
        pl.kernel,
        out_type=jax.ShapeDtypeStruct((NPAD, D), jnp.float32),
        mesh=mesh,
        scratch_types=[
            pltpu.VMEM((NB, NG, RG), jnp.int32),
            pltpu.VMEM((2, NG, RG, D), jnp.float32),
            pltpu.VMEM((2, BN, D), jnp.float32),
            pltpu.SemaphoreType.DMA,
            pltpu.SemaphoreType.DMA,
        ],
    )
    def _sc_gather_sum(x_hbm, idx_hbm, out_hbm, idx_v, rows_v, acc_v,
                       sem0, sem1):
        wid = lax.axis_index("s") * NC + lax.axis_index("c")
        pltpu.sync_copy(idx_hbm.at[wid], idx_v)
        sems = [sem0, sem1]

        def fire(b, p):
            for g in range(NG):
                pltpu.async_copy(x_hbm.at[idx_v.at[b, g]],
                                 rows_v.at[p, g], sems[p])

        def drain(p):
            for g in range(NG):
                pltpu.make_async_copy(x_hbm.at[pl.ds(0, RG)],
                                      rows_v.at[p, g], sems[p]).wait()

        def accum_write(b, p):
            for i in range(BN):
                for v in range(D // L):
                    r0 = i * K
                    s = rows_v[p, r0 // RG, r0 % RG, pl.ds(v * L, L)]
                    for k in range(1, K):
                        r = r0 + k
                        s = s + rows_v[p, r // RG, r % RG, pl.ds(v * L, L)]
                    acc_v[p, i, pl.ds(v * L, L)] = s
            pltpu.sync_copy(acc_v.at[p],
                            out_hbm.at[pl.ds(wid * PW + b * BN, BN)])

        fire(0, 0)

        def body(j, carry):
            b0 = 2 * j
            fire(b0 + 1, 1)
            drain(0)
            accum_write(b0, 0)

            @pl.when(j < NB // 2 - 1)
            def _():
                fire(b0 + 2, 0)

            drain(1)
            accum_write(b0 + 1, 1)
            return carry

        lax.fori_loop(0, NB // 2, body, 0)

    return _sc_gather_sum


def _tc1_body(ts_ref, et_ref, ef_ref, x_ref, wt_ref, bt_ref,
              w1b_ref, w1cr_ref, b1_ref, w2b_ref, w2c_ref, b2_ref,
              a_ref, p_ref):
    w = wt_ref[...]
    b = bt_ref[...]
    dt = ts_ref[...] - et_ref[...]
    s_te = jnp.cos(dt[:, 0][:, None] * w + b)
    for k in range(1, K):
        s_te = s_te + jnp.cos(dt[:, k][:, None] * w + b)
    a = lax.dot_general(s_te, w1b_ref[...], (((1,), (0,)), ((), ())),
                        preferred_element_type=jnp.float32)
    a += lax.dot_general(ef_ref[...], w1cr_ref[...], (((1,), (0,)), ((), ())),
                         preferred_element_type=jnp.float32)
    a_ref[...] = a + K * b1_ref[...]
    p = lax.dot_general(x_ref[...], w2b_ref[...], (((1,), (0,)), ((), ())),
                        preferred_element_type=jnp.float32)
    src = jnp.cos(b)
    p += lax.dot_general(src, w2c_ref[...], (((1,), (0,)), ((), ())),
                         preferred_element_type=jnp.float32)
    p_ref[...] = p + b2_ref[...]


def _tc1(ts2, et, ef_flat, x, wt2, bt2, w1b, w1cr, b12, w2b, w2c, b22,
         interpret=False):
    B = 256
    grid = (pl.cdiv(N, B),)
    row = lambda i: (i, 0)
    full = lambda i: (0, 0)
    return pl.pallas_call(
        _tc1_body,
        grid=grid,
        in_specs=[
            pl.BlockSpec((B, 1), row),
            pl.BlockSpec((B, K), row),
            pl.BlockSpec((B, K * DE), row),
            pl.BlockSpec((B, D), row),
            pl.BlockSpec((1, D), full),
            pl.BlockSpec((1, D), full),
            pl.BlockSpec((D, D), full),
            pl.BlockSpec((K * DE, D), full),
            pl.BlockSpec((1, D), full),
            pl.BlockSpec((D, D), full),
            pl.BlockSpec((D, D), full),
            pl.BlockSpec((1, D), full),
        ],
        out_specs=[pl.BlockSpec((B, D), row), pl.BlockSpec((B, D), row)],
        out_shape=[jax.ShapeDtypeStruct((N, D), jnp.float32),
                   jax.ShapeDtypeStruct((N, D), jnp.float32)],
        interpret=interpret,
    )(ts2, et, ef_flat, x, wt2, bt2, w1b, w1cr, b12, w2b, w2c, b22)


def _tc2_body(snf_ref, a_ref, p_ref, w1a_ref, w2a_ref, out_ref):
    acc = lax.dot_general(snf_ref[...], w1a_ref[...], (((1,), (0,)), ((), ())),
                          preferred_element_type=jnp.float32)
    h = jnp.maximum(acc + a_ref[...], 0.0)
    out = lax.dot_general(h, w2a_ref[...], (((1,), (0,)), ((), ())),
                          preferred_element_type=jnp.float32)
    out_ref[...] = out + p_ref[...]


def _tc2(snf, a, p, w1a, w2a, interpret=False):
    B = 2048
    grid = (pl.cdiv(N, B),)
    row = lambda i: (i, 0)
    full = lambda i: (0, 0)
    return pl.pallas_call(
        _tc2_body,
        grid=grid,
        in_specs=[
            pl.BlockSpec((B, D), row),
            pl.BlockSpec((B, D), row),
            pl.BlockSpec((B, D), row),
            pl.BlockSpec((D, D), full),
            pl.BlockSpec((D, D), full),
        ],
        out_specs=pl.BlockSpec((B, D), row),
        out_shape=jax.ShapeDtypeStruct((N, D), jnp.float32),
        interpret=interpret,
    )(snf, a, p, w1a, w2a)


def kernel(x, timestamps, edge_times, edge_feat, neighbors, w_t, b_t,
           W1, b1, W2, b2):
    nbr = neighbors.astype(jnp.int32)
    nbr_pad = jnp.pad(nbr, ((0, NPAD - N), (0, 0)))
    idx3 = nbr_pad.reshape(NW, NB, NG, RG)
    snf = _make_sc_gather_sum()(x, idx3)

    ts2 = timestamps.reshape(N, 1)
    ef_flat = edge_feat.reshape(N, K * DE)
    w1a, w1b, w1c = W1[:D], W1[D:2 * D], W1[2 * D:]
    w1cr = jnp.tile(w1c, (K, 1))
    w2a, w2b, w2c = W2[:D], W2[D:2 * D], W2[2 * D:]
    a, p = _tc1(ts2, edge_times, ef_flat, x,
                w_t.reshape(1, D), b_t.reshape(1, D),
                w1b, w1cr, b1.reshape(1, D),
                w2b, w2c, b2.reshape(1, D))
    return _tc2(snf, a, p, w1a, w2a)
